# Initial kernel scaffold; baseline (speedup 1.0000x reference)
import jax
import jax.numpy as jnp
from jax.experimental import pallas as pl


def kernel(x, sigma, centroids, boundaries):
    raise NotImplementedError("write your pallas kernel here")


# speedup vs baseline: 13.6247x; 13.6247x over previous
import functools
import math

import jax
import jax.numpy as jnp
import numpy as np
from jax.experimental import pallas as pl
from jax.experimental.pallas import tpu as pltpu

_D = 1024
_CH = 128
_NCH = _D // _CH
_BN = 512


def _had128_bf16():
    i = np.arange(_CH)
    pc = np.array([bin(v).count("1") for v in range(_CH)])
    signs = 1.0 - 2.0 * (pc[(i[:, None] & i[None, :])] % 2)
    return jnp.asarray(signs, dtype=jnp.bfloat16)


def _bfly8(t):
    c = [t[:, k * _CH:(k + 1) * _CH] for k in range(_NCH)]
    d = [c[0] + c[4], c[1] + c[5], c[2] + c[6], c[3] + c[7],
         c[0] - c[4], c[1] - c[5], c[2] - c[6], c[3] - c[7]]
    e = [d[0] + d[2], d[1] + d[3], d[0] - d[2], d[1] - d[3],
         d[4] + d[6], d[5] + d[7], d[4] - d[6], d[5] - d[7]]
    f = [e[0] + e[1], e[0] - e[1], e[2] + e[3], e[2] - e[3],
         e[4] + e[5], e[4] - e[5], e[6] + e[7], e[6] - e[7]]
    return jnp.concatenate(f, axis=1)


def _mm2(a, h):
    ah = a.astype(jnp.bfloat16)
    al = (a - ah.astype(jnp.float32)).astype(jnp.bfloat16)
    return (jnp.dot(ah, h, preferred_element_type=jnp.float32)
            + jnp.dot(al, h, preferred_element_type=jnp.float32))


def _tq_kernel(x_ref, sig_ref, h_ref, c_ref, b_ref, xhat_ref, idx_ref):
    h = h_ref[...]
    t = x_ref[...] * sig_ref[...]
    t = _bfly8(t)
    y = _mm2(t.reshape(_BN * _NCH, _CH), h)

    idx = jnp.zeros(y.shape, dtype=jnp.int32)
    yh = jnp.full(y.shape, c_ref[0], dtype=jnp.float32)
    for k in range(15):
        cmp = y > (b_ref[k] * 32.0)
        idx = idx + cmp.astype(jnp.int32)
        yh = yh + jnp.where(cmp, c_ref[k + 1] - c_ref[k], 0.0)

    v = _mm2(yh, h).reshape(_BN, _D)
    v = _bfly8(v)
    xhat_ref[...] = v * (sig_ref[...] * (1.0 / 32.0))
    idx_ref[...] = idx.reshape(_BN, _D)


@jax.jit
def kernel(x, sigma, centroids, boundaries):
    n = x.shape[0]
    h = _had128_bf16()
    grid = (n // _BN,)
    x_hat, idx = pl.pallas_call(
        _tq_kernel,
        grid=grid,
        in_specs=[
            pl.BlockSpec((_BN, _D), lambda i: (i, 0)),
            pl.BlockSpec((1, _D), lambda i: (0, 0)),
            pl.BlockSpec((_CH, _CH), lambda i: (0, 0)),
            pl.BlockSpec(memory_space=pltpu.SMEM),
            pl.BlockSpec(memory_space=pltpu.SMEM),
        ],
        out_specs=[
            pl.BlockSpec((_BN, _D), lambda i: (i, 0)),
            pl.BlockSpec((_BN, _D), lambda i: (i, 0)),
        ],
        out_shape=[
            jax.ShapeDtypeStruct((n, _D), jnp.float32),
            jax.ShapeDtypeStruct((n, _D), jnp.int32),
        ],
    )(x, sigma.reshape(1, _D), h, centroids, boundaries)
    return (x_hat, idx)


# speedup vs baseline: 19.4943x; 1.4308x over previous
import functools
import math

import jax
import jax.numpy as jnp
import numpy as np
from jax.experimental import pallas as pl
from jax.experimental.pallas import tpu as pltpu

_D = 1024
_CH = 128
_NCH = _D // _CH
_BN = 512


def _had128_bf16():
    i = np.arange(_CH)
    pc = np.array([bin(v).count("1") for v in range(_CH)])
    signs = 1.0 - 2.0 * (pc[(i[:, None] & i[None, :])] % 2)
    return jnp.asarray(signs, dtype=jnp.bfloat16)


def _bfly8(t):
    c = [t[:, k * _CH:(k + 1) * _CH] for k in range(_NCH)]
    d = [c[0] + c[4], c[1] + c[5], c[2] + c[6], c[3] + c[7],
         c[0] - c[4], c[1] - c[5], c[2] - c[6], c[3] - c[7]]
    e = [d[0] + d[2], d[1] + d[3], d[0] - d[2], d[1] - d[3],
         d[4] + d[6], d[5] + d[7], d[4] - d[6], d[5] - d[7]]
    f = [e[0] + e[1], e[0] - e[1], e[2] + e[3], e[2] - e[3],
         e[4] + e[5], e[4] - e[5], e[6] + e[7], e[6] - e[7]]
    return jnp.concatenate(f, axis=1)


def _mm2(a, h):
    ah = a.astype(jnp.bfloat16)
    al = (a - ah.astype(jnp.float32)).astype(jnp.bfloat16)
    return (jnp.dot(ah, h, preferred_element_type=jnp.float32)
            + jnp.dot(al, h, preferred_element_type=jnp.float32))


def _tq_kernel(x_ref, sig_ref, h_ref, c_ref, b_ref, xhat_ref, idx_ref):
    h = h_ref[...]
    t = x_ref[...] * sig_ref[...]
    t = _bfly8(t)
    y = _mm2(t.reshape(_BN * _NCH, _CH), h)

    b = [b_ref[k] * 32.0 for k in range(15)]
    w = jnp.where
    m3 = y > b[7]
    m2 = y > w(m3, b[11], b[3])
    m1 = y > w(m3, w(m2, b[13], b[9]), w(m2, b[5], b[1]))
    m0 = y > w(m3,
               w(m2, w(m1, b[14], b[12]), w(m1, b[10], b[8])),
               w(m2, w(m1, b[6], b[4]), w(m1, b[2], b[0])))
    p16 = [64.0 * k + c_ref[k] for k in range(16)]
    q = [w(m0, p16[2 * j + 1], p16[2 * j]) for j in range(8)]
    q = [w(m1, q[2 * j + 1], q[2 * j]) for j in range(4)]
    q = [w(m2, q[2 * j + 1], q[2 * j]) for j in range(2)]
    p = w(m3, q[1], q[0])
    idxf = jnp.round(p * (1.0 / 64.0))
    yh = p - idxf * 64.0

    v = jnp.dot(yh.astype(jnp.bfloat16), h,
                preferred_element_type=jnp.float32).reshape(_BN, _D)
    v = _bfly8(v)
    xhat_ref[...] = v * (sig_ref[...] * (1.0 / 32.0))
    idx_ref[...] = idxf.astype(jnp.int32).reshape(_BN, _D)


@jax.jit
def kernel(x, sigma, centroids, boundaries):
    n = x.shape[0]
    h = _had128_bf16()
    grid = (n // _BN,)
    x_hat, idx = pl.pallas_call(
        _tq_kernel,
        grid=grid,
        in_specs=[
            pl.BlockSpec((_BN, _D), lambda i: (i, 0)),
            pl.BlockSpec((1, _D), lambda i: (0, 0)),
            pl.BlockSpec((_CH, _CH), lambda i: (0, 0)),
            pl.BlockSpec(memory_space=pltpu.SMEM),
            pl.BlockSpec(memory_space=pltpu.SMEM),
        ],
        out_specs=[
            pl.BlockSpec((_BN, _D), lambda i: (i, 0)),
            pl.BlockSpec((_BN, _D), lambda i: (i, 0)),
        ],
        out_shape=[
            jax.ShapeDtypeStruct((n, _D), jnp.float32),
            jax.ShapeDtypeStruct((n, _D), jnp.int32),
        ],
    )(x, sigma.reshape(1, _D), h, centroids, boundaries)
    return (x_hat, idx)
